# baseline (speedup 1.0000x reference)
import jax
import jax.numpy as jnp
from jax import lax
from jax.experimental import pallas as pl
from jax.experimental.pallas import tpu as pltpu
from jax.experimental.pallas import tpu_sc as plsc

N_NODES = 10000
D = 128
DH = D // 2

NC = 2
NS = 16

LANES = 128
G_TOTAL = 2560
G0_PER_TILE = 160
G1_PER_TILE = 0
G_STAGE = 160
NBUF = 4

N_PAD = 10240
ROWS_PER_TILE = N_PAD // NS
WB_CHUNKS = ROWS_PER_TILE // LANES
IDX_ROWS = NS * G0_PER_TILE + (NS - 1) * G1_PER_TILE + G_STAGE


def _mm1_body(x_ref, w_ref, olo_ref, ohi_ref):
    r = jnp.dot(x_ref[...], w_ref[...], preferred_element_type=jnp.float32)
    olo_ref[...] = r[:, :DH]
    ohi_ref[...] = r[:, DH:]


def _mm1(x, W):
    BM = 400
    return pl.pallas_call(
        _mm1_body,
        grid=(N_NODES // BM,),
        in_specs=[
            pl.BlockSpec((BM, D), lambda i: (i, 0)),
            pl.BlockSpec((D, D), lambda i: (0, 0)),
        ],
        out_specs=[
            pl.BlockSpec((BM, DH), lambda i: (i, 0)),
            pl.BlockSpec((BM, DH), lambda i: (i, 0)),
        ],
        out_shape=[
            jax.ShapeDtypeStruct((N_NODES, DH), jnp.float32),
            jax.ShapeDtypeStruct((N_NODES, DH), jnp.float32),
        ],
    )(x, W)


def _mm2_body(p0lo_ref, p1lo_ref, p0hi_ref, p1hi_ref, b_ref, w_ref,
              olo_ref, ohi_ref):
    h = jnp.concatenate(
        [p0lo_ref[...] + p1lo_ref[...], p0hi_ref[...] + p1hi_ref[...]],
        axis=1)
    h = jnp.maximum(h + b_ref[...], 0.0)
    r = jnp.dot(h, w_ref[...], preferred_element_type=jnp.float32)
    olo_ref[...] = r[:, :DH]
    ohi_ref[...] = r[:, DH:]


def _mm2(plo, phi, b, W):
    BM = 512
    nblk = N_PAD // BM
    return pl.pallas_call(
        _mm2_body,
        grid=(nblk,),
        in_specs=[
            pl.BlockSpec((BM, DH), lambda i: (i, 0)),
            pl.BlockSpec((BM, DH), lambda i: (i + nblk, 0)),
            pl.BlockSpec((BM, DH), lambda i: (i, 0)),
            pl.BlockSpec((BM, DH), lambda i: (i + nblk, 0)),
            pl.BlockSpec((1, D), lambda i: (0, 0)),
            pl.BlockSpec((D, D), lambda i: (0, 0)),
        ],
        out_specs=[
            pl.BlockSpec((BM, DH), lambda i: (i, 0)),
            pl.BlockSpec((BM, DH), lambda i: (i, 0)),
        ],
        out_shape=[
            jax.ShapeDtypeStruct((N_PAD, DH), jnp.float32),
            jax.ShapeDtypeStruct((N_PAD, DH), jnp.float32),
        ],
    )(plo, plo, phi, phi, b.reshape(1, D), W)


def _final_body(q0lo_ref, q1lo_ref, q0hi_ref, q1hi_ref, b_ref, o_ref):
    o_ref[...] = jnp.concatenate(
        [q0lo_ref[...] + q1lo_ref[...], q0hi_ref[...] + q1hi_ref[...]],
        axis=1) + b_ref[...]


def _final(qlo, qhi, b):
    BM = 80
    return pl.pallas_call(
        _final_body,
        grid=(N_NODES // BM,),
        in_specs=[
            pl.BlockSpec((BM, DH), lambda i: (i, 0)),
            pl.BlockSpec((BM, DH), lambda i: (i + N_PAD // BM, 0)),
            pl.BlockSpec((BM, DH), lambda i: (i, 0)),
            pl.BlockSpec((BM, DH), lambda i: (i + N_PAD // BM, 0)),
            pl.BlockSpec((1, D), lambda i: (0, 0)),
        ],
        out_specs=pl.BlockSpec((BM, D), lambda i: (i, 0)),
        out_shape=jax.ShapeDtypeStruct((N_NODES, D), jnp.float32),
    )(qlo, qlo, qhi, qhi, b.reshape(1, D))


def _sc_agg_body(suplo_hbm, suphi_hbm, src_hbm, dst_hbm,
                 outlo_hbm, outhi_hbm,
                 src_v, dst_v,
                 rows0, rows1, rows2, rows3,
                 acc_sh,
                 sem0, sem1, sem2, sem3):
    cid = lax.axis_index("c")
    sid = lax.axis_index("s")
    bufs = [(rows0, sem0), (rows1, sem1), (rows2, sem2), (rows3, sem3)]
    row0 = sid * ROWS_PER_TILE
    out_base = cid * N_PAD + row0

    gbase = jnp.where(cid == 0, sid * G0_PER_TILE,
                      NS * G0_PER_TILE + sid * G1_PER_TILE)
    n_rounds = jnp.where(cid == 0, G0_PER_TILE // NBUF,
                         G1_PER_TILE // NBUF)
    n_groups = n_rounds * NBUF

    with jax.named_scope("ph_stage"):
        pltpu.sync_copy(src_hbm.at[pl.ds(gbase, G_STAGE)], src_v)
        pltpu.sync_copy(dst_hbm.at[pl.ds(gbase, G_STAGE)], dst_v)

    zero16 = jnp.zeros((16,), jnp.float32)

    for hf, (sup_hbm, out_hbm) in enumerate(
            ((suplo_hbm, outlo_hbm), (suphi_hbm, outhi_hbm))):
        with jax.named_scope(f"ph_fill{hf}"):
            def _zrow(r, carry):
                def _zcol(c, carry2):
                    rows0[r, pl.ds(c * 16, 16)] = zero16
                    return carry2
                return lax.fori_loop(0, DH // 16, _zcol, carry)

            lax.fori_loop(0, LANES, _zrow, 0)

        with jax.named_scope(f"ph_zero{hf}"):
            def _zcp(m, carry):
                pltpu.sync_copy(rows0,
                                acc_sh.at[pl.ds(row0 + m * LANES, LANES)])
                return carry

            lax.fori_loop(0, WB_CHUNKS, _zcp, 0)
            plsc.subcore_barrier()

        with jax.named_scope(f"ph_edges{hf}"):
            @pl.when(n_groups > 0)
            def _(sup_hbm=sup_hbm):
                for j, (rb, sb) in enumerate(bufs):
                    pltpu.async_copy(sup_hbm.at[src_v.at[j]], rb, sb)

            def _round(t, carry):
                g0 = t * NBUF
                for j, (rb, sb) in enumerate(bufs):
                    g = g0 + j
                    pltpu.make_async_copy(sup_hbm.at[src_v.at[g]], rb,
                                          sb).wait()
                    pltpu.sync_copy(rb, acc_sh.at[dst_v.at[g]], add=True)

                    @pl.when(g + NBUF < n_groups)
                    def _(rb=rb, sb=sb, g=g, sup_hbm=sup_hbm):
                        pltpu.async_copy(sup_hbm.at[src_v.at[g + NBUF]],
                                         rb, sb)
                return carry

            lax.fori_loop(0, n_rounds, _round, 0)
            plsc.subcore_barrier()

        with jax.named_scope(f"ph_wb{hf}"):
            def _wb(m, carry):
                pltpu.sync_copy(acc_sh.at[pl.ds(row0 + m * LANES, LANES)],
                                rows0)
                pltpu.sync_copy(
                    rows0, out_hbm.at[pl.ds(out_base + m * LANES, LANES)])
                return carry

            lax.fori_loop(0, WB_CHUNKS, _wb, 0)
            plsc.subcore_barrier()


def _sc_agg(sup_lo, sup_hi, src2d, dst2d):
    mesh = plsc.VectorSubcoreMesh(core_axis_name="c", subcore_axis_name="s",
                                  num_cores=NC, num_subcores=NS)
    kern = pl.kernel(
        _sc_agg_body,
        out_type=[
            jax.ShapeDtypeStruct((NC * N_PAD, DH), jnp.float32),
            jax.ShapeDtypeStruct((NC * N_PAD, DH), jnp.float32),
        ],
        mesh=mesh,
        compiler_params=pltpu.CompilerParams(use_tc_tiling_on_sc=False),
        scratch_types=(
            [pltpu.VMEM((G_STAGE, LANES), jnp.int32)] * 2
            + [pltpu.VMEM((LANES, DH), jnp.float32)] * NBUF
            + [pltpu.VMEM_SHARED((N_PAD, DH), jnp.float32)]
            + [pltpu.SemaphoreType.DMA] * NBUF
        ),
    )
    return kern(sup_lo, sup_hi, src2d, dst2d)


def kernel(x, adj, W1, b1, W2, b2):
    src = adj[0].astype(jnp.int32)
    dst = adj[1].astype(jnp.int32)
    n_edges = src.shape[0]
    pad = IDX_ROWS * LANES - n_edges
    src_p = jnp.concatenate(
        [src, jnp.zeros((pad,), jnp.int32)]).reshape(IDX_ROWS, LANES)
    dst_p = jnp.concatenate(
        [dst, jnp.full((pad,), N_NODES, jnp.int32)]).reshape(
            IDX_ROWS, LANES)

    s1lo, s1hi = _mm1(x, W1)
    p1lo, p1hi = _sc_agg(s1lo, s1hi, src_p, dst_p)
    s2lo, s2hi = _mm2(p1lo, p1hi, b1, W2)
    p2lo, p2hi = _sc_agg(s2lo, s2hi, src_p, dst_p)
    return _final(p2lo, p2hi, b2)


# speedup vs baseline: 3.2358x; 3.2358x over previous
import jax
import jax.numpy as jnp
from jax import lax
from jax.experimental import pallas as pl
from jax.experimental.pallas import tpu as pltpu
from jax.experimental.pallas import tpu_sc as plsc

N_NODES = 10000
D = 128
DH = D // 2

NC = 2
NS = 16

LANES = 128
G_TOTAL = 2560
G0_PER_TILE = 80
G1_PER_TILE = 80
G_STAGE = 80
NBUF = 4

N_PAD = 10240
ROWS_PER_TILE = N_PAD // NS
WB_CHUNKS = ROWS_PER_TILE // LANES
IDX_ROWS = NS * G0_PER_TILE + (NS - 1) * G1_PER_TILE + G_STAGE


def _mm1_body(x_ref, w_ref, olo_ref, ohi_ref):
    r = jnp.dot(x_ref[...], w_ref[...], preferred_element_type=jnp.float32)
    olo_ref[...] = r[:, :DH]
    ohi_ref[...] = r[:, DH:]


def _mm1(x, W):
    BM = 400
    return pl.pallas_call(
        _mm1_body,
        grid=(N_NODES // BM,),
        in_specs=[
            pl.BlockSpec((BM, D), lambda i: (i, 0)),
            pl.BlockSpec((D, D), lambda i: (0, 0)),
        ],
        out_specs=[
            pl.BlockSpec((BM, DH), lambda i: (i, 0)),
            pl.BlockSpec((BM, DH), lambda i: (i, 0)),
        ],
        out_shape=[
            jax.ShapeDtypeStruct((N_NODES, DH), jnp.float32),
            jax.ShapeDtypeStruct((N_NODES, DH), jnp.float32),
        ],
    )(x, W)


def _mm2_body(p0lo_ref, p1lo_ref, p0hi_ref, p1hi_ref, b_ref, w_ref,
              olo_ref, ohi_ref):
    h = jnp.concatenate(
        [p0lo_ref[...] + p1lo_ref[...], p0hi_ref[...] + p1hi_ref[...]],
        axis=1)
    h = jnp.maximum(h + b_ref[...], 0.0)
    r = jnp.dot(h, w_ref[...], preferred_element_type=jnp.float32)
    olo_ref[...] = r[:, :DH]
    ohi_ref[...] = r[:, DH:]


def _mm2(plo, phi, b, W):
    BM = 512
    nblk = N_PAD // BM
    return pl.pallas_call(
        _mm2_body,
        grid=(nblk,),
        in_specs=[
            pl.BlockSpec((BM, DH), lambda i: (i, 0)),
            pl.BlockSpec((BM, DH), lambda i: (i + nblk, 0)),
            pl.BlockSpec((BM, DH), lambda i: (i, 0)),
            pl.BlockSpec((BM, DH), lambda i: (i + nblk, 0)),
            pl.BlockSpec((1, D), lambda i: (0, 0)),
            pl.BlockSpec((D, D), lambda i: (0, 0)),
        ],
        out_specs=[
            pl.BlockSpec((BM, DH), lambda i: (i, 0)),
            pl.BlockSpec((BM, DH), lambda i: (i, 0)),
        ],
        out_shape=[
            jax.ShapeDtypeStruct((N_PAD, DH), jnp.float32),
            jax.ShapeDtypeStruct((N_PAD, DH), jnp.float32),
        ],
    )(plo, plo, phi, phi, b.reshape(1, D), W)


def _final_body(q0lo_ref, q1lo_ref, q0hi_ref, q1hi_ref, b_ref, o_ref):
    o_ref[...] = jnp.concatenate(
        [q0lo_ref[...] + q1lo_ref[...], q0hi_ref[...] + q1hi_ref[...]],
        axis=1) + b_ref[...]


def _final(qlo, qhi, b):
    BM = 80
    return pl.pallas_call(
        _final_body,
        grid=(N_NODES // BM,),
        in_specs=[
            pl.BlockSpec((BM, DH), lambda i: (i, 0)),
            pl.BlockSpec((BM, DH), lambda i: (i + N_PAD // BM, 0)),
            pl.BlockSpec((BM, DH), lambda i: (i, 0)),
            pl.BlockSpec((BM, DH), lambda i: (i + N_PAD // BM, 0)),
            pl.BlockSpec((1, D), lambda i: (0, 0)),
        ],
        out_specs=pl.BlockSpec((BM, D), lambda i: (i, 0)),
        out_shape=jax.ShapeDtypeStruct((N_NODES, D), jnp.float32),
    )(qlo, qlo, qhi, qhi, b.reshape(1, D))


def _sc_agg_body(suplo_hbm, suphi_hbm, src_hbm, dst_hbm,
                 outlo_hbm, outhi_hbm,
                 src_v, dst_v,
                 rows0, rows1, rows2, rows3,
                 acc_sh,
                 sem0, sem1, sem2, sem3):
    cid = lax.axis_index("c")
    sid = lax.axis_index("s")
    bufs = [(rows0, sem0), (rows1, sem1), (rows2, sem2), (rows3, sem3)]
    row0 = sid * ROWS_PER_TILE
    out_base = cid * N_PAD + row0

    gbase = jnp.where(cid == 0, sid * G0_PER_TILE,
                      NS * G0_PER_TILE + sid * G1_PER_TILE)
    n_rounds = jnp.where(cid == 0, G0_PER_TILE // NBUF,
                         G1_PER_TILE // NBUF)
    n_groups = n_rounds * NBUF

    with jax.named_scope("ph_stage"):
        pltpu.sync_copy(src_hbm.at[pl.ds(gbase, G_STAGE)], src_v)
        pltpu.sync_copy(dst_hbm.at[pl.ds(gbase, G_STAGE)], dst_v)

    zero16 = jnp.zeros((16,), jnp.float32)

    for hf, (sup_hbm, out_hbm) in enumerate(
            ((suplo_hbm, outlo_hbm), (suphi_hbm, outhi_hbm))):
        with jax.named_scope(f"ph_fill{hf}"):
            def _zrow(r, carry):
                def _zcol(c, carry2):
                    rows0[r, pl.ds(c * 16, 16)] = zero16
                    return carry2
                return lax.fori_loop(0, DH // 16, _zcol, carry)

            lax.fori_loop(0, LANES, _zrow, 0)

        with jax.named_scope(f"ph_zero{hf}"):
            def _zcp(m, carry):
                pltpu.sync_copy(rows0,
                                acc_sh.at[pl.ds(row0 + m * LANES, LANES)])
                return carry

            lax.fori_loop(0, WB_CHUNKS, _zcp, 0)
            plsc.subcore_barrier()

        with jax.named_scope(f"ph_edges{hf}"):
            @pl.when(n_groups > 0)
            def _(sup_hbm=sup_hbm):
                for j, (rb, sb) in enumerate(bufs):
                    pltpu.async_copy(sup_hbm.at[src_v.at[j]], rb, sb)

            def _round(t, carry):
                g0 = t * NBUF
                for j, (rb, sb) in enumerate(bufs):
                    g = g0 + j
                    pltpu.make_async_copy(sup_hbm.at[src_v.at[g]], rb,
                                          sb).wait()
                    pltpu.sync_copy(rb, acc_sh.at[dst_v.at[g]], add=True)

                    @pl.when(g + NBUF < n_groups)
                    def _(rb=rb, sb=sb, g=g, sup_hbm=sup_hbm):
                        pltpu.async_copy(sup_hbm.at[src_v.at[g + NBUF]],
                                         rb, sb)
                return carry

            lax.fori_loop(0, n_rounds, _round, 0)
            plsc.subcore_barrier()

        with jax.named_scope(f"ph_wb{hf}"):
            def _wb(m, carry):
                pltpu.sync_copy(acc_sh.at[pl.ds(row0 + m * LANES, LANES)],
                                rows0)
                pltpu.sync_copy(
                    rows0, out_hbm.at[pl.ds(out_base + m * LANES, LANES)])
                return carry

            lax.fori_loop(0, WB_CHUNKS, _wb, 0)
            plsc.subcore_barrier()


def _sc_agg(sup_lo, sup_hi, src2d, dst2d):
    mesh = plsc.VectorSubcoreMesh(core_axis_name="c", subcore_axis_name="s",
                                  num_cores=NC, num_subcores=NS)
    kern = pl.kernel(
        _sc_agg_body,
        out_type=[
            jax.ShapeDtypeStruct((NC * N_PAD, DH), jnp.float32),
            jax.ShapeDtypeStruct((NC * N_PAD, DH), jnp.float32),
        ],
        mesh=mesh,
        compiler_params=pltpu.CompilerParams(use_tc_tiling_on_sc=False),
        scratch_types=(
            [pltpu.VMEM((G_STAGE, LANES), jnp.int32)] * 2
            + [pltpu.VMEM((LANES, DH), jnp.float32)] * NBUF
            + [pltpu.VMEM_SHARED((N_PAD, DH), jnp.float32)]
            + [pltpu.SemaphoreType.DMA] * NBUF
        ),
    )
    return kern(sup_lo, sup_hi, src2d, dst2d)


def kernel(x, adj, W1, b1, W2, b2):
    src = adj[0].astype(jnp.int32)
    dst = adj[1].astype(jnp.int32)
    n_edges = src.shape[0]
    pad = IDX_ROWS * LANES - n_edges
    pad_idx = jnp.arange(pad, dtype=jnp.int32)
    src_p = jnp.concatenate(
        [src, pad_idx % N_NODES]).reshape(IDX_ROWS, LANES)
    dst_p = jnp.concatenate(
        [dst, N_NODES + pad_idx % (N_PAD - N_NODES)]).reshape(
            IDX_ROWS, LANES)

    s1lo, s1hi = _mm1(x, W1)
    p1lo, p1hi = _sc_agg(s1lo, s1hi, src_p, dst_p)
    s2lo, s2hi = _mm2(p1lo, p1hi, b1, W2)
    p2lo, p2hi = _sc_agg(s2lo, s2hi, src_p, dst_p)
    return _final(p2lo, p2hi, b2)
